# baseline (speedup 1.0000x reference)
import functools

import jax
import jax.numpy as jnp
from jax import lax
from jax.experimental import pallas as pl
from jax.experimental.pallas import tpu as pltpu
from jax.experimental.pallas import tpu_sc as plsc

NUM_NODES = 10000
NUM_EDGES = 320000
D = 128

NC, NS = 2, 16
NW = NC * NS
E_W = NUM_EDGES // NW
K = 64
NSTAGE = 8
CPS = 20
E_WP = NSTAGE * CPS * K
PAD = NW * E_WP - NUM_EDGES
ACC_ROWS = 10240
ROWS_PER_TILE = ACC_ROWS // NS


def _sc_aggregate(node_x, src4, tgt4):
    mesh = plsc.VectorSubcoreMesh(core_axis_name="c", subcore_axis_name="s")

    @functools.partial(
        pl.kernel,
        out_type=jax.ShapeDtypeStruct((NC * ACC_ROWS, D), jnp.float32),
        mesh=mesh,
        scratch_types=[
            pltpu.VMEM((CPS, K), jnp.int32),
            pltpu.VMEM((CPS, K), jnp.int32),
            pltpu.VMEM((K, D), jnp.float32),
            pltpu.VMEM((K, D), jnp.float32),
            pltpu.VMEM((K, D), jnp.float32),
            pltpu.VMEM((K, D), jnp.float32),
            pltpu.VMEM_SHARED((ACC_ROWS, D), jnp.float32),
            pltpu.SemaphoreType.DMA,
            pltpu.SemaphoreType.DMA,
            pltpu.SemaphoreType.DMA,
            pltpu.SemaphoreType.DMA,
        ],
    )
    def agg_kernel(node_hbm, src_hbm, tgt_hbm, out_hbm,
                   src_v, tgt_v, rows_a, rows_b, rows_c, rows_d, acc_sh,
                   gsem_a, gsem_b, gsem_c, gsem_d):
        cid = lax.axis_index("c")
        sid = lax.axis_index("s")
        wid = sid * NC + cid

        def zrow(r, carry):
            for c16 in range(D // 16):
                rows_a[r, pl.ds(c16 * 16, 16)] = jnp.zeros((16,), jnp.float32)
            return carry
        lax.fori_loop(0, K, zrow, 0)
        for t in range(ROWS_PER_TILE // K):
            pltpu.sync_copy(
                rows_a, acc_sh.at[pl.ds(sid * ROWS_PER_TILE + t * K, K)])
        plsc.subcore_barrier()

        def gather(j, buf, sem):
            return pltpu.async_copy(node_hbm.at[src_v.at[j]], buf, sem)

        def scatter(j, buf, sem):
            return pltpu.async_copy(buf, acc_sh.at[tgt_v.at[j]], sem,
                                    add=True)

        def wait_gather(j, buf, sem):
            pltpu.make_async_copy(node_hbm.at[src_v.at[j]], buf, sem).wait()

        def wait_scatter(j, buf, sem):
            pltpu.make_async_copy(buf, acc_sh.at[tgt_v.at[j]], sem).wait()

        NBUF = 4
        bufs = ((rows_a, gsem_a), (rows_b, gsem_b), (rows_c, gsem_c),
                (rows_d, gsem_d))

        def step(j, slot, issue_next):
            buf, sem = bufs[slot]
            wait_gather(j, buf, sem)
            pltpu.sync_copy(buf, acc_sh.at[tgt_v.at[j]], add=True)
            if issue_next:
                gather(j + NBUF, buf, sem)

        for s in range(NSTAGE):
            pltpu.sync_copy(src_hbm.at[wid, s], src_v)
            pltpu.sync_copy(tgt_hbm.at[wid, s], tgt_v)
            for p in range(NBUF):
                gather(p, *bufs[p])

            def group(i, carry):
                j = NBUF * i
                for p in range(NBUF):
                    step(j + p, p, True)
                return carry
            lax.fori_loop(0, (CPS - 2 * NBUF) // NBUF, group, 0)

            for j in range(CPS - 2 * NBUF, CPS):
                step(j, j % NBUF, j + NBUF < CPS)
        plsc.subcore_barrier()
        plsc.subcore_barrier()

        base = cid * ACC_ROWS + sid * ROWS_PER_TILE
        pltpu.sync_copy(
            acc_sh.at[pl.ds(sid * ROWS_PER_TILE, ROWS_PER_TILE)],
            out_hbm.at[pl.ds(base, ROWS_PER_TILE)])

    return agg_kernel(node_x, src4, tgt4)


def _dense(partials, node_x, W, b2):
    BR = 1000

    def body(p_ref, x_ref, w_ref, b_ref, o_ref):
        agg = p_ref[0] + p_ref[1]
        acc = jnp.dot(agg, w_ref[:D, :], preferred_element_type=jnp.float32)
        acc += jnp.dot(x_ref[...], w_ref[D:, :],
                       preferred_element_type=jnp.float32)
        o_ref[...] = acc + b_ref[...]

    return pl.pallas_call(
        body,
        grid=(NUM_NODES // BR,),
        in_specs=[
            pl.BlockSpec((2, BR, D), lambda i: (0, i, 0)),
            pl.BlockSpec((BR, D), lambda i: (i, 0)),
            pl.BlockSpec((2 * D, D), lambda i: (0, 0)),
            pl.BlockSpec((1, D), lambda i: (0, 0)),
        ],
        out_specs=pl.BlockSpec((BR, D), lambda i: (i, 0)),
        out_shape=jax.ShapeDtypeStruct((NUM_NODES, D), jnp.float32),
    )(partials, node_x, W, b2)


def kernel(node_x, edge_x, sources, targets, features, W, b):
    del edge_x, features
    src_p = sources.astype(jnp.int32)
    tgt_p = targets.astype(jnp.int32)
    if PAD:
        src_p = jnp.concatenate([src_p, jnp.zeros((PAD,), jnp.int32)])
        tgt_p = jnp.concatenate(
            [tgt_p, NUM_NODES + jnp.arange(PAD, dtype=jnp.int32)
             % (ACC_ROWS - NUM_NODES)])
    src4 = src_p.reshape(NW, NSTAGE, CPS, K)
    tgt4 = tgt_p.reshape(NW, NSTAGE, CPS, K)
    partials = _sc_aggregate(node_x, src4, tgt4).reshape(NC, ACC_ROWS, D)
    return _dense(partials, node_x, W, b.reshape(1, D))


# speedup vs baseline: 3.8756x; 3.8756x over previous
import functools

import jax
import jax.numpy as jnp
from jax import lax
from jax.experimental import pallas as pl
from jax.experimental.pallas import tpu as pltpu
from jax.experimental.pallas import tpu_sc as plsc

NUM_NODES = 10000
NUM_EDGES = 320000
D = 128

NC, NS = 2, 16
NW = NC * NS
E_W = NUM_EDGES // NW
K = 80
NSTAGE = 5
CPS = 25
E_WP = NSTAGE * CPS * K
PAD = NW * E_WP - NUM_EDGES
ACC_ROWS = 10240
ROWS_PER_TILE = ACC_ROWS // NS


def _sc_aggregate(node_x, src4, tgt4):
    mesh = plsc.VectorSubcoreMesh(core_axis_name="c", subcore_axis_name="s")

    @functools.partial(
        pl.kernel,
        out_type=jax.ShapeDtypeStruct((NC * ACC_ROWS, D), jnp.float32),
        mesh=mesh,
        scratch_types=[
            pltpu.VMEM((CPS, K), jnp.int32),
            pltpu.VMEM((CPS, K), jnp.int32),
            pltpu.VMEM((CPS, K), jnp.int32),
            pltpu.VMEM((CPS, K), jnp.int32),
            pltpu.VMEM((K, D), jnp.float32),
            pltpu.VMEM((K, D), jnp.float32),
            pltpu.VMEM((K, D), jnp.float32),
            pltpu.VMEM_SHARED((ACC_ROWS, D), jnp.float32),
            pltpu.SemaphoreType.DMA,
            pltpu.SemaphoreType.DMA,
            pltpu.SemaphoreType.DMA,
            pltpu.SemaphoreType.DMA,
        ],
    )
    def agg_kernel(node_hbm, src_hbm, tgt_hbm, out_hbm,
                   src_v0, tgt_v0, src_v1, tgt_v1,
                   rows_a, rows_b, rows_c, acc_sh,
                   gsem_a, gsem_b, gsem_c, isem):
        cid = lax.axis_index("c")
        sid = lax.axis_index("s")
        wid = sid * NC + cid

        def zrow(r, carry):
            for c16 in range(D // 16):
                rows_a[r, pl.ds(c16 * 16, 16)] = jnp.zeros((16,), jnp.float32)
            return carry
        lax.fori_loop(0, K, zrow, 0)
        for t in range(ROWS_PER_TILE // K):
            pltpu.sync_copy(
                rows_a, acc_sh.at[pl.ds(sid * ROWS_PER_TILE + t * K, K)])
        plsc.subcore_barrier()

        def gather(sv, j, buf, sem):
            return pltpu.async_copy(node_hbm.at[sv.at[j]], buf, sem)

        def wait_gather(sv, j, buf, sem):
            pltpu.make_async_copy(node_hbm.at[sv.at[j]], buf, sem).wait()

        bufs = ((rows_a, gsem_a), (rows_b, gsem_b), (rows_c, gsem_c))
        slabs = ((src_v0, tgt_v0), (src_v1, tgt_v1))

        def step(sv, tv, j, slot, nxt):
            buf, sem = bufs[slot]
            wait_gather(sv, j, buf, sem)
            pltpu.sync_copy(buf, acc_sh.at[tv.at[j]], add=True)
            if nxt is not None:
                nsv, nj = nxt
                gather(nsv, nj, buf, sem)

        pltpu.sync_copy(src_hbm.at[wid, 0], src_v0)
        pltpu.sync_copy(tgt_hbm.at[wid, 0], tgt_v0)
        pltpu.async_copy(src_hbm.at[wid, 1], src_v1, isem)
        pltpu.async_copy(tgt_hbm.at[wid, 1], tgt_v1, isem)
        for p in range(3):
            gather(src_v0, p, *bufs[p])

        for s in range(NSTAGE):
            sv, tv = slabs[s % 2]
            nsv, ntv = slabs[(s + 1) % 2]

            def triple(i, carry):
                j = 3 * i
                for p in range(3):
                    step(sv, tv, j + p, (p + s) % 3, (sv, j + p + 3))
                return carry
            lax.fori_loop(0, (CPS - 4) // 3, triple, 0)

            step(sv, tv, CPS - 4, (CPS - 4 + s) % 3, (sv, CPS - 1))
            last = s == NSTAGE - 1
            if not last:
                pltpu.make_async_copy(src_hbm.at[wid, s + 1], nsv,
                                      isem).wait()
                pltpu.make_async_copy(tgt_hbm.at[wid, s + 1], ntv,
                                      isem).wait()
            for q in range(3):
                j = CPS - 3 + q
                step(sv, tv, j, (j + s) % 3,
                     None if last else (nsv, q))
            if s + 2 < NSTAGE:
                pltpu.async_copy(src_hbm.at[wid, s + 2], sv, isem)
                pltpu.async_copy(tgt_hbm.at[wid, s + 2], tv, isem)
        plsc.subcore_barrier()
        plsc.subcore_barrier()

        base = cid * ACC_ROWS + sid * ROWS_PER_TILE
        pltpu.sync_copy(
            acc_sh.at[pl.ds(sid * ROWS_PER_TILE, ROWS_PER_TILE)],
            out_hbm.at[pl.ds(base, ROWS_PER_TILE)])

    return agg_kernel(node_x, src4, tgt4)


def _dense(partials, node_x, W, b2):
    BR = 1000

    def body(p_ref, x_ref, w_ref, b_ref, o_ref):
        agg = p_ref[0] + p_ref[1]
        acc = jnp.dot(agg, w_ref[:D, :], preferred_element_type=jnp.float32)
        acc += jnp.dot(x_ref[...], w_ref[D:, :],
                       preferred_element_type=jnp.float32)
        o_ref[...] = acc + b_ref[...]

    return pl.pallas_call(
        body,
        grid=(NUM_NODES // BR,),
        in_specs=[
            pl.BlockSpec((2, BR, D), lambda i: (0, i, 0)),
            pl.BlockSpec((BR, D), lambda i: (i, 0)),
            pl.BlockSpec((2 * D, D), lambda i: (0, 0)),
            pl.BlockSpec((1, D), lambda i: (0, 0)),
        ],
        out_specs=pl.BlockSpec((BR, D), lambda i: (i, 0)),
        out_shape=jax.ShapeDtypeStruct((NUM_NODES, D), jnp.float32),
    )(partials, node_x, W, b2)


def kernel(node_x, edge_x, sources, targets, features, W, b):
    del edge_x, features
    src_p = sources.astype(jnp.int32)
    tgt_p = targets.astype(jnp.int32)
    if PAD:
        src_p = jnp.concatenate([src_p, jnp.zeros((PAD,), jnp.int32)])
        tgt_p = jnp.concatenate(
            [tgt_p, NUM_NODES + jnp.arange(PAD, dtype=jnp.int32)
             % (ACC_ROWS - NUM_NODES)])
    src4 = src_p.reshape(NW, NSTAGE, CPS, K)
    tgt4 = tgt_p.reshape(NW, NSTAGE, CPS, K)
    partials = _sc_aggregate(node_x, src4, tgt4).reshape(NC, ACC_ROWS, D)
    return _dense(partials, node_x, W, b.reshape(1, D))


# speedup vs baseline: 3.9555x; 1.0206x over previous
import functools

import jax
import jax.numpy as jnp
from jax import lax
from jax.experimental import pallas as pl
from jax.experimental.pallas import tpu as pltpu
from jax.experimental.pallas import tpu_sc as plsc

NUM_NODES = 10000
NUM_EDGES = 320000
D = 128

NC, NS = 2, 16
NW = NC * NS
E_W = NUM_EDGES // NW
K = 80
NSTAGE = 5
CPS = 25
E_WP = NSTAGE * CPS * K
PAD = NW * E_WP - NUM_EDGES
ACC_ROWS = 10240
ROWS_PER_TILE = ACC_ROWS // NS


def _sc_aggregate(node_x, src4, tgt4):
    mesh = plsc.VectorSubcoreMesh(core_axis_name="c", subcore_axis_name="s")

    @functools.partial(
        pl.kernel,
        out_type=jax.ShapeDtypeStruct((NC * ACC_ROWS, D), jnp.float32),
        mesh=mesh,
        scratch_types=[
            pltpu.VMEM((CPS, K), jnp.int32),
            pltpu.VMEM((CPS, K), jnp.int32),
            pltpu.VMEM((CPS, K), jnp.int32),
            pltpu.VMEM((CPS, K), jnp.int32),
            pltpu.VMEM((K, D), jnp.float32),
            pltpu.VMEM((K, D), jnp.float32),
            pltpu.VMEM((K, D), jnp.float32),
            pltpu.VMEM_SHARED((ACC_ROWS, D), jnp.float32),
            pltpu.SemaphoreType.DMA,
            pltpu.SemaphoreType.DMA,
            pltpu.SemaphoreType.DMA,
            pltpu.SemaphoreType.DMA,
        ],
    )
    def agg_kernel(node_hbm, src_hbm, tgt_hbm, out_hbm,
                   src_v0, tgt_v0, src_v1, tgt_v1,
                   rows_a, rows_b, rows_c, acc_sh,
                   gsem_a, gsem_b, gsem_c, isem):
        cid = lax.axis_index("c")
        sid = lax.axis_index("s")
        wid = sid * NC + cid

        pltpu.async_copy(src_hbm.at[wid, 0], src_v0, isem)
        pltpu.async_copy(tgt_hbm.at[wid, 0], tgt_v0, isem)
        pltpu.async_copy(src_hbm.at[wid, 1], src_v1, isem)
        pltpu.async_copy(tgt_hbm.at[wid, 1], tgt_v1, isem)

        def zrow(r, carry):
            for c16 in range(D // 16):
                rows_a[r, pl.ds(c16 * 16, 16)] = jnp.zeros((16,), jnp.float32)
            return carry
        lax.fori_loop(0, K, zrow, 0)
        for t in range(ROWS_PER_TILE // K):
            pltpu.async_copy(
                rows_a, acc_sh.at[pl.ds(sid * ROWS_PER_TILE + t * K, K)],
                gsem_a)
        for t in range(ROWS_PER_TILE // K):
            pltpu.make_async_copy(
                rows_a, acc_sh.at[pl.ds(sid * ROWS_PER_TILE + t * K, K)],
                gsem_a).wait()
        plsc.subcore_barrier()

        def gather(sv, j, buf, sem):
            return pltpu.async_copy(node_hbm.at[sv.at[j]], buf, sem)

        def wait_gather(sv, j, buf, sem):
            pltpu.make_async_copy(node_hbm.at[sv.at[j]], buf, sem).wait()

        bufs = ((rows_a, gsem_a), (rows_b, gsem_b), (rows_c, gsem_c))
        slabs = ((src_v0, tgt_v0), (src_v1, tgt_v1))

        def step(sv, tv, j, slot, nxt):
            buf, sem = bufs[slot]
            wait_gather(sv, j, buf, sem)
            pltpu.sync_copy(buf, acc_sh.at[tv.at[j]], add=True)
            if nxt is not None:
                nsv, nj = nxt
                gather(nsv, nj, buf, sem)

        pltpu.make_async_copy(src_hbm.at[wid, 0], src_v0, isem).wait()
        pltpu.make_async_copy(tgt_hbm.at[wid, 0], tgt_v0, isem).wait()
        for p in range(3):
            gather(src_v0, p, *bufs[p])

        for s in range(NSTAGE):
            sv, tv = slabs[s % 2]
            nsv, ntv = slabs[(s + 1) % 2]

            def triple(i, carry):
                j = 3 * i
                for p in range(3):
                    step(sv, tv, j + p, (p + s) % 3, (sv, j + p + 3))
                return carry
            lax.fori_loop(0, (CPS - 4) // 3, triple, 0)

            step(sv, tv, CPS - 4, (CPS - 4 + s) % 3, (sv, CPS - 1))
            last = s == NSTAGE - 1
            if not last:
                pltpu.make_async_copy(src_hbm.at[wid, s + 1], nsv,
                                      isem).wait()
                pltpu.make_async_copy(tgt_hbm.at[wid, s + 1], ntv,
                                      isem).wait()
            for q in range(3):
                j = CPS - 3 + q
                step(sv, tv, j, (j + s) % 3,
                     None if last else (nsv, q))
            if s + 2 < NSTAGE:
                pltpu.async_copy(src_hbm.at[wid, s + 2], sv, isem)
                pltpu.async_copy(tgt_hbm.at[wid, s + 2], tv, isem)
        plsc.subcore_barrier()
        plsc.subcore_barrier()

        base = cid * ACC_ROWS + sid * ROWS_PER_TILE
        pltpu.sync_copy(
            acc_sh.at[pl.ds(sid * ROWS_PER_TILE, ROWS_PER_TILE)],
            out_hbm.at[pl.ds(base, ROWS_PER_TILE)])

    return agg_kernel(node_x, src4, tgt4)


def _dense(partials, node_x, W, b2):
    BR = 1000

    def body(p_ref, x_ref, w_ref, b_ref, o_ref):
        agg = p_ref[0] + p_ref[1]
        acc = jnp.dot(agg, w_ref[:D, :], preferred_element_type=jnp.float32)
        acc += jnp.dot(x_ref[...], w_ref[D:, :],
                       preferred_element_type=jnp.float32)
        o_ref[...] = acc + b_ref[...]

    return pl.pallas_call(
        body,
        grid=(NUM_NODES // BR,),
        in_specs=[
            pl.BlockSpec((2, BR, D), lambda i: (0, i, 0)),
            pl.BlockSpec((BR, D), lambda i: (i, 0)),
            pl.BlockSpec((2 * D, D), lambda i: (0, 0)),
            pl.BlockSpec((1, D), lambda i: (0, 0)),
        ],
        out_specs=pl.BlockSpec((BR, D), lambda i: (i, 0)),
        out_shape=jax.ShapeDtypeStruct((NUM_NODES, D), jnp.float32),
    )(partials, node_x, W, b2)


def kernel(node_x, edge_x, sources, targets, features, W, b):
    del edge_x, features
    src_p = sources.astype(jnp.int32)
    tgt_p = targets.astype(jnp.int32)
    if PAD:
        src_p = jnp.concatenate([src_p, jnp.zeros((PAD,), jnp.int32)])
        tgt_p = jnp.concatenate(
            [tgt_p, NUM_NODES + jnp.arange(PAD, dtype=jnp.int32)
             % (ACC_ROWS - NUM_NODES)])
    src4 = src_p.reshape(NW, NSTAGE, CPS, K)
    tgt4 = tgt_p.reshape(NW, NSTAGE, CPS, K)
    partials = _sc_aggregate(node_x, src4, tgt4).reshape(NC, ACC_ROWS, D)
    return _dense(partials, node_x, W, b.reshape(1, D))
